# Initial kernel scaffold; baseline (speedup 1.0000x reference)
import jax
import jax.numpy as jnp
from jax.experimental import pallas as pl


def kernel(state, action, W_t, b_t, A_emb, W_p):
    raise NotImplementedError("write your pallas kernel here")


# speedup vs baseline: 1.2371x; 1.2371x over previous
import jax
import jax.numpy as jnp
from jax.experimental import pallas as pl

_B, _D, _A = 2048, 128, 512
_MAX_T = 17
_HORIZON = 2
_BLOCK = 256
_NB = _B // _BLOCK


def _rollout_body(state_ref, act_ref, wt_ref, bt_ref, aemb_ref, wp_ref, out_ref):
    f32 = jnp.float32
    state = state_ref[...]
    act = act_ref[...]
    W_t = wt_ref[...]
    b_t = bt_ref[...]
    A_emb = aemb_ref[...]
    W_p = wp_ref[...]

    lane_d = jax.lax.broadcasted_iota(jnp.int32, (_BLOCK, _D), 1)
    lane_a = jax.lax.broadcasted_iota(jnp.int32, (_BLOCK, _A), 1)
    p_sel = lane_d < 2
    w_sel = (lane_d >= 2) & (lane_d < 5)

    def argfirst(x, iota, width):
        mx = jnp.max(x, axis=1, keepdims=True)
        return jnp.min(jnp.where(x == mx, iota, width), axis=1, keepdims=True)

    def player_winner(s):
        sp = jnp.where(p_sel, s, f32(-1.0))
        sw = jnp.where(w_sel, s, f32(-1.0))
        return argfirst(sp, lane_d, _D), argfirst(sw, lane_d, _D)

    def onehot_emb(idx):
        oh = (lane_a == idx).astype(f32)
        return jnp.dot(oh, A_emb, preferred_element_type=f32)

    emb0 = onehot_emb(act)

    def transition(s, emb):
        return jax.nn.sigmoid(
            jnp.dot(s, W_t, preferred_element_type=f32) + emb + b_t)

    for _step in range(_HORIZON):
        init_p, init_w = player_winner(state)
        state = transition(state, emb0)

        def in_progress(s):
            cp, cw = player_winner(s)
            return ((cp != init_p) & (cw == init_w)).astype(jnp.int32)

        def cond(c):
            t, _s, m = c
            return (t < _MAX_T) & (jnp.max(m) > 0)

        def body(c):
            t, s, m = c
            p_logits = jnp.dot(s, W_p, preferred_element_type=f32)
            aidx = argfirst(p_logits, lane_a, _A)
            ns = transition(s, onehot_emb(aidx))
            s2 = jnp.where(m > 0, ns, s)
            return t + 1, s2, in_progress(s2)

        m0 = in_progress(state)
        _, state, _ = jax.lax.while_loop(cond, body, (0, state, m0))

    out_ref[...] = state


@jax.jit
def kernel(state, action, W_t, b_t, A_emb, W_p):
    act2 = action.reshape(_B, 1)
    bt2 = b_t.reshape(1, _D)
    return pl.pallas_call(
        _rollout_body,
        grid=(_NB,),
        in_specs=[
            pl.BlockSpec((_BLOCK, _D), lambda i: (i, 0)),
            pl.BlockSpec((_BLOCK, 1), lambda i: (i, 0)),
            pl.BlockSpec((_D, _D), lambda i: (0, 0)),
            pl.BlockSpec((1, _D), lambda i: (0, 0)),
            pl.BlockSpec((_A, _D), lambda i: (0, 0)),
            pl.BlockSpec((_D, _A), lambda i: (0, 0)),
        ],
        out_specs=pl.BlockSpec((_BLOCK, _D), lambda i: (i, 0)),
        out_shape=jax.ShapeDtypeStruct((_B, _D), jnp.float32),
    )(state, act2, W_t, bt2, A_emb, W_p)


# speedup vs baseline: 4.0752x; 3.2941x over previous
import jax
import jax.numpy as jnp
from jax.experimental import pallas as pl
from jax.experimental.pallas import tpu as pltpu

_B, _D, _A = 2048, 128, 512
_MAX_T = 17
_HORIZON = 2
_G = 256
_NG = _B // _G
_C = 256
_R = _B // 128


def _rollout_body(state_ref, act_ref, wt_ref, bt_ref, aemb_ref, wp_ref,
                  out_ref, m0_ref, ip_ref, iw_ref):
    f32 = jnp.float32
    i32 = jnp.int32
    W_t = wt_ref[...]
    b_t = bt_ref[...]
    A_emb = aemb_ref[...]
    W_p = wp_ref[...]

    lane_d = jax.lax.broadcasted_iota(i32, (_G, _D), 1)
    p_sel = lane_d < 2
    w_sel = (lane_d >= 2) & (lane_d < 5)
    lane_a = jax.lax.broadcasted_iota(i32, (_G, _A), 1)
    eye128 = (jax.lax.broadcasted_iota(i32, (128, 128), 0) ==
              jax.lax.broadcasted_iota(i32, (128, 128), 1)).astype(f32)
    cum_u = (jax.lax.broadcasted_iota(i32, (128, 128), 0) <=
             jax.lax.broadcasted_iota(i32, (128, 128), 1)).astype(f32)
    excl_l = (jax.lax.broadcasted_iota(i32, (_R, _R), 1) <
              jax.lax.broadcasted_iota(i32, (_R, _R), 0)).astype(f32)
    ones128 = jnp.ones((128, 1), f32)
    ones_c = jnp.ones((_C, 1), f32)
    k_iota = jax.lax.broadcasted_iota(i32, (_C, 128), 0).astype(f32)

    def argfirst(x, iota, width):
        mx = jnp.max(x, axis=1, keepdims=True)
        return jnp.min(jnp.where(x == mx, iota, width), axis=1, keepdims=True)

    def player_winner(s):
        n = s.shape[0]
        ld = lane_d[:n]
        sp = jnp.where(p_sel[:n], s, f32(-1.0))
        sw = jnp.where(w_sel[:n], s, f32(-1.0))
        return argfirst(sp, ld, _D), argfirst(sw, ld, _D)

    def onehot_emb(idx, n):
        oh = (lane_a[:n] == idx).astype(f32)
        return jnp.dot(oh, A_emb, preferred_element_type=f32)

    def transition(s, emb):
        return jax.nn.sigmoid(
            jnp.dot(s, W_t, preferred_element_type=f32) + emb + b_t)

    for _step in range(_HORIZON):
        src = state_ref if _step == 0 else out_ref

        for g in range(_NG):
            rows = pl.ds(g * _G, _G)
            s = src[rows, :]
            ip, iw = player_winner(s)
            act = act_ref[rows, :]
            s = transition(s, onehot_emb(act, _G))
            cp, cw = player_winner(s)
            m0 = ((cp != ip) & (cw == iw)).astype(f32)
            out_ref[rows, :] = s
            ip_ref[rows, :] = ip.astype(f32)
            iw_ref[rows, :] = iw.astype(f32)
            m0_ref[rows, :] = m0

        m_rows = [
            jax.lax.dot_general(m0_ref[pl.ds(r * 128, 128), :], eye128,
                                (((0,), (0,)), ((), ())),
                                preferred_element_type=f32)
            for r in range(_R)
        ]
        m_lane = jnp.concatenate(m_rows, axis=0)
        c_within = jnp.dot(m_lane, cum_u, preferred_element_type=f32)
        tot = jnp.dot(m_lane, ones128, preferred_element_type=f32)
        offs = jax.lax.dot_general(excl_l, tot, (((1,), (0,)), ((), ())),
                                   preferred_element_type=f32)
        pos_lane = c_within + offs - 1.0
        n_act = jnp.sum(tot).astype(i32)

        def make_onehots(start):
            s_f = start.astype(f32)
            ohs = []
            for r in range(_R):
                rel = pos_lane[r:r + 1, :] - s_f
                hit = (k_iota == rel) & (m_lane[r:r + 1, :] > 0.0)
                ohs.append(hit.astype(f32))
            return ohs

        def gather(ohs, ref, width):
            acc = None
            for r in range(_R):
                blk = ref[pl.ds(r * 128, 128), :]
                p = jnp.dot(ohs[r], blk, preferred_element_type=f32)
                acc = p if acc is None else acc + p
            return acc

        def chunk_body(start):
            ohs = make_onehots(start)
            comp = gather(ohs, out_ref, _D)
            ipc = gather(ohs, ip_ref, 1).astype(i32)
            iwc = gather(ohs, iw_ref, 1).astype(i32)

            def in_progress(s):
                cp, cw = player_winner(s)
                return ((cp != ipc) & (cw == iwc)).astype(i32)

            def cond(c):
                t, _s, m = c
                return (t < _MAX_T) & (jnp.max(m) > 0)

            def body(c):
                t, s, m = c
                p_logits = jnp.dot(s, W_p, preferred_element_type=f32)
                aidx = argfirst(p_logits, lane_a[:_C], _A)
                ns = transition(s, onehot_emb(aidx, _C))
                s2 = jnp.where(m > 0, ns, s)
                return t + 1, s2, in_progress(s2)

            _, comp, _ = jax.lax.while_loop(
                cond, body, (jnp.int32(0), comp, in_progress(comp)))

            for r in range(_R):
                rows = pl.ds(r * 128, 128)
                scat = jax.lax.dot_general(ohs[r], comp,
                                           (((0,), (0,)), ((), ())),
                                           preferred_element_type=f32)
                memb = jax.lax.dot_general(ohs[r], ones_c,
                                           (((0,), (0,)), ((), ())),
                                           preferred_element_type=f32)
                out_ref[rows, :] = jnp.where(memb > 0.0, scat,
                                             out_ref[rows, :])
            return start + _C

        jax.lax.while_loop(lambda s: s < n_act, chunk_body, jnp.int32(0))


@jax.jit
def kernel(state, action, W_t, b_t, A_emb, W_p):
    act2 = action.reshape(_B, 1)
    bt2 = b_t.reshape(1, _D)
    return pl.pallas_call(
        _rollout_body,
        grid=(1,),
        in_specs=[
            pl.BlockSpec((_B, _D), lambda i: (0, 0)),
            pl.BlockSpec((_B, 1), lambda i: (0, 0)),
            pl.BlockSpec((_D, _D), lambda i: (0, 0)),
            pl.BlockSpec((1, _D), lambda i: (0, 0)),
            pl.BlockSpec((_A, _D), lambda i: (0, 0)),
            pl.BlockSpec((_D, _A), lambda i: (0, 0)),
        ],
        out_specs=pl.BlockSpec((_B, _D), lambda i: (0, 0)),
        out_shape=jax.ShapeDtypeStruct((_B, _D), jnp.float32),
        scratch_shapes=[
            pltpu.VMEM((_B, 1), jnp.float32),
            pltpu.VMEM((_B, 1), jnp.float32),
            pltpu.VMEM((_B, 1), jnp.float32),
        ],
    )(state, act2, W_t, bt2, A_emb, W_p)


# speedup vs baseline: 4.5959x; 1.1278x over previous
import jax
import jax.numpy as jnp
from jax.experimental import pallas as pl
from jax.experimental.pallas import tpu as pltpu

_B, _D, _A = 2048, 128, 512
_MAX_T = 17
_HORIZON = 2
_G = 256
_NG = _B // _G
_C = 256
_R = _B // 128


def _rollout_body(state_ref, act_ref, wt_ref, bt_ref, aemb_ref, wp_ref,
                  out_ref, m0_ref, code_ref, emb0_ref):
    f32 = jnp.float32
    i32 = jnp.int32
    W_t = wt_ref[...]
    b_t = bt_ref[...]
    A_emb = aemb_ref[...]
    W_p = wp_ref[...]

    lane_a = jax.lax.broadcasted_iota(i32, (_G, _A), 1)
    eye128 = (jax.lax.broadcasted_iota(i32, (128, 128), 0) ==
              jax.lax.broadcasted_iota(i32, (128, 128), 1)).astype(f32)
    cum_u = (jax.lax.broadcasted_iota(i32, (128, 128), 0) <=
             jax.lax.broadcasted_iota(i32, (128, 128), 1)).astype(f32)
    excl_l = (jax.lax.broadcasted_iota(i32, (_R, _R), 1) <
              jax.lax.broadcasted_iota(i32, (_R, _R), 0)).astype(f32)
    ones128 = jnp.ones((128, 1), f32)
    ones_c = jnp.ones((_C, 1), f32)
    k_iota = jax.lax.broadcasted_iota(i32, (_C, 128), 0).astype(f32)

    def argfirst(x, iota, width):
        mx = jnp.max(x, axis=1, keepdims=True)
        return jnp.min(jnp.where(x == mx, iota, width), axis=1, keepdims=True)

    def pw_code(s):
        p = (s[:, 1:2] > s[:, 0:1]).astype(f32)
        a, b, c = s[:, 2:3], s[:, 3:4], s[:, 4:5]
        w1 = (b > a) & (b >= c)
        w2 = (c > a) & (c > b)
        w = jnp.where(w1, f32(1.0), f32(0.0)) + jnp.where(w2, f32(2.0),
                                                          f32(0.0))
        return 4.0 * p + w

    def in_progress_f(code_now, code_init):
        return (jnp.abs(code_now - code_init) == 4.0).astype(i32)

    def onehot_emb(idx, n):
        oh = (lane_a[:n] == idx).astype(f32)
        return jnp.dot(oh, A_emb, preferred_element_type=f32)

    def transition(s, emb):
        return jax.nn.sigmoid(
            jnp.dot(s, W_t, preferred_element_type=f32) + emb + b_t)

    for g in range(_NG):
        rows = pl.ds(g * _G, _G)
        emb0_ref[rows, :] = onehot_emb(act_ref[rows, :], _G)

    for _step in range(_HORIZON):
        src = state_ref if _step == 0 else out_ref

        for g in range(_NG):
            rows = pl.ds(g * _G, _G)
            s = src[rows, :]
            code_i = pw_code(s)
            s = transition(s, emb0_ref[rows, :])
            m0 = in_progress_f(pw_code(s), code_i).astype(f32)
            out_ref[rows, :] = s
            code_ref[rows, :] = code_i
            m0_ref[rows, :] = m0

        m_rows = [
            jax.lax.dot_general(m0_ref[pl.ds(r * 128, 128), :], eye128,
                                (((0,), (0,)), ((), ())),
                                preferred_element_type=f32)
            for r in range(_R)
        ]
        m_lane = jnp.concatenate(m_rows, axis=0)
        c_within = jnp.dot(m_lane, cum_u, preferred_element_type=f32)
        tot = jnp.dot(m_lane, ones128, preferred_element_type=f32)
        offs = jax.lax.dot_general(excl_l, tot, (((1,), (0,)), ((), ())),
                                   preferred_element_type=f32)
        pos_lane = c_within + offs - 1.0
        n_act = jnp.sum(tot).astype(i32)

        def make_onehots(start):
            s_f = start.astype(f32)
            ohs = []
            for r in range(_R):
                rel = pos_lane[r:r + 1, :] - s_f
                hit = (k_iota == rel) & (m_lane[r:r + 1, :] > 0.0)
                ohs.append(hit.astype(f32))
            return ohs

        def gather(ohs, ref, width):
            acc = None
            for r in range(_R):
                blk = ref[pl.ds(r * 128, 128), :]
                p = jnp.dot(ohs[r], blk, preferred_element_type=f32)
                acc = p if acc is None else acc + p
            return acc

        def chunk_body(start):
            ohs = make_onehots(start)
            comp = gather(ohs, out_ref, _D)
            code_c = gather(ohs, code_ref, 1)

            def cond(c):
                t, _s, m = c
                return (t < _MAX_T) & (jnp.max(m) > 0)

            def body(c):
                t, s, m = c
                p_logits = jnp.dot(s, W_p, preferred_element_type=f32)
                aidx = argfirst(p_logits, lane_a[:_C], _A)
                ns = transition(s, onehot_emb(aidx, _C))
                s2 = jnp.where(m > 0, ns, s)
                return t + 1, s2, in_progress_f(pw_code(s2), code_c)

            m_init = in_progress_f(pw_code(comp), code_c)
            _, comp, _ = jax.lax.while_loop(
                cond, body, (jnp.int32(0), comp, m_init))

            for r in range(_R):
                rows = pl.ds(r * 128, 128)
                scat = jax.lax.dot_general(ohs[r], comp,
                                           (((0,), (0,)), ((), ())),
                                           preferred_element_type=f32)
                memb = jax.lax.dot_general(ohs[r], ones_c,
                                           (((0,), (0,)), ((), ())),
                                           preferred_element_type=f32)
                out_ref[rows, :] = jnp.where(memb > 0.0, scat,
                                             out_ref[rows, :])
            return start + _C

        jax.lax.while_loop(lambda s: s < n_act, chunk_body, jnp.int32(0))


@jax.jit
def kernel(state, action, W_t, b_t, A_emb, W_p):
    act2 = action.reshape(_B, 1)
    bt2 = b_t.reshape(1, _D)
    return pl.pallas_call(
        _rollout_body,
        grid=(1,),
        in_specs=[
            pl.BlockSpec((_B, _D), lambda i: (0, 0)),
            pl.BlockSpec((_B, 1), lambda i: (0, 0)),
            pl.BlockSpec((_D, _D), lambda i: (0, 0)),
            pl.BlockSpec((1, _D), lambda i: (0, 0)),
            pl.BlockSpec((_A, _D), lambda i: (0, 0)),
            pl.BlockSpec((_D, _A), lambda i: (0, 0)),
        ],
        out_specs=pl.BlockSpec((_B, _D), lambda i: (0, 0)),
        out_shape=jax.ShapeDtypeStruct((_B, _D), jnp.float32),
        scratch_shapes=[
            pltpu.VMEM((_B, 1), jnp.float32),
            pltpu.VMEM((_B, 1), jnp.float32),
            pltpu.VMEM((_B, _D), jnp.float32),
        ],
    )(state, act2, W_t, bt2, A_emb, W_p)


# speedup vs baseline: 6.6092x; 1.4381x over previous
import jax
import jax.numpy as jnp
from jax.experimental import pallas as pl
from jax.experimental.pallas import tpu as pltpu

_B, _D, _A = 2048, 128, 512
_MAX_T = 17
_HORIZON = 2
_CG = 512
_NCG = _B // _CG
_C = 256
_R = _B // 128


def _rollout_body(state_ref, act_ref, wt_ref, bt_ref, aemb_ref, wp_ref,
                  out_ref, st_ref, emb0_ref):
    f32 = jnp.float32
    i32 = jnp.int32
    W_t = wt_ref[...]
    b_t1 = bt_ref[...]
    A_emb = aemb_ref[...]
    W_p = wp_ref[...]

    eye128 = (jax.lax.broadcasted_iota(i32, (128, 128), 0) ==
              jax.lax.broadcasted_iota(i32, (128, 128), 1)).astype(f32)
    cum_u = (jax.lax.broadcasted_iota(i32, (128, 128), 0) <=
             jax.lax.broadcasted_iota(i32, (128, 128), 1)).astype(f32)
    excl_l = (jax.lax.broadcasted_iota(i32, (_R, _R), 1) <
              jax.lax.broadcasted_iota(i32, (_R, _R), 0)).astype(f32)
    ones128 = jnp.ones((128, 1), f32)
    ones1c = jnp.ones((1, _C), f32)
    k_iota = jax.lax.broadcasted_iota(i32, (_C, 128), 0).astype(f32)
    sub_ac = jax.lax.broadcasted_iota(i32, (_A, _C), 0)
    sub_ag = jax.lax.broadcasted_iota(i32, (_A, _CG), 0)

    def tr(x):
        return jax.lax.dot_general(x, eye128, (((0,), (0,)), ((), ())),
                                   preferred_element_type=f32)

    def argfirst0(x, iota, width):
        mx = jnp.max(x, axis=0, keepdims=True)
        return jnp.min(jnp.where(x == mx, iota, width), axis=0,
                       keepdims=True)

    def pw_code(sT):
        p = (sT[1:2, :] > sT[0:1, :]).astype(f32)
        a, b, c = sT[2:3, :], sT[3:4, :], sT[4:5, :]
        w1 = (b > a) & (b >= c)
        w2 = (c > a) & (c > b)
        w = jnp.where(w1, f32(1.0), f32(0.0)) + jnp.where(w2, f32(2.0),
                                                          f32(0.0))
        return 4.0 * p + w

    def in_progress_f(code_now, code_init):
        return (jnp.abs(code_now - code_init) == 4.0).astype(i32)

    def transition(sT, embT):
        return jax.nn.sigmoid(
            jax.lax.dot_general(W_t, sT, (((0,), (0,)), ((), ())),
                                preferred_element_type=f32) + embT + b_t1)

    for r in range(_R):
        st_ref[:, pl.ds(r * 128, 128)] = tr(state_ref[pl.ds(r * 128, 128), :])
    for g in range(_NCG):
        cols = pl.ds(g * _CG, _CG)
        oh0 = (sub_ag == act_ref[:, cols]).astype(f32)
        emb0_ref[:, cols] = jax.lax.dot_general(
            A_emb, oh0, (((0,), (0,)), ((), ())), preferred_element_type=f32)

    for _step in range(_HORIZON):
        code_gs, m_gs = [], []
        for g in range(_NCG):
            cols = pl.ds(g * _CG, _CG)
            sT = st_ref[:, cols]
            code_g = pw_code(sT)
            sT = transition(sT, emb0_ref[:, cols])
            m_gs.append(in_progress_f(pw_code(sT), code_g).astype(f32))
            st_ref[:, cols] = sT
            code_gs.append(code_g)
        code_full = jnp.concatenate(code_gs, axis=1)
        m_full = jnp.concatenate(m_gs, axis=1)

        m_lane = jnp.concatenate(
            [m_full[:, r * 128:(r + 1) * 128] for r in range(_R)], axis=0)
        c_within = jnp.dot(m_lane, cum_u, preferred_element_type=f32)
        tot = jnp.dot(m_lane, ones128, preferred_element_type=f32)
        offs = jax.lax.dot_general(excl_l, tot, (((1,), (0,)), ((), ())),
                                   preferred_element_type=f32)
        pos_lane = c_within + offs - 1.0
        n_act = jnp.sum(tot).astype(i32)

        def make_onehots(start):
            s_f = start.astype(f32)
            ohs = []
            for r in range(_R):
                rel = pos_lane[r:r + 1, :] - s_f
                hit = (k_iota == rel) & (m_lane[r:r + 1, :] > 0.0)
                ohs.append(hit.astype(f32))
            return ohs

        def chunk_body(start):
            ohs = make_onehots(start)
            comp = None
            code_c = None
            for r in range(_R):
                cols = pl.ds(r * 128, 128)
                pc = jax.lax.dot_general(
                    st_ref[:, cols], ohs[r], (((1,), (1,)), ((), ())),
                    preferred_element_type=f32)
                cc = jax.lax.dot_general(
                    code_full[:, r * 128:(r + 1) * 128], ohs[r],
                    (((1,), (1,)), ((), ())), preferred_element_type=f32)
                comp = pc if comp is None else comp + pc
                code_c = cc if code_c is None else code_c + cc

            def cond(c):
                t, _s, m = c
                return (t < _MAX_T) & (jnp.max(m) > 0)

            def body(c):
                t, sT, m = c
                p_logits = jax.lax.dot_general(
                    W_p, sT, (((0,), (0,)), ((), ())),
                    preferred_element_type=f32)
                aidx = argfirst0(p_logits, sub_ac, _A)
                oh = (sub_ac == aidx).astype(f32)
                emb = jax.lax.dot_general(
                    A_emb, oh, (((0,), (0,)), ((), ())),
                    preferred_element_type=f32)
                ns = transition(sT, emb)
                s2 = jnp.where(m > 0, ns, sT)
                return t + 1, s2, in_progress_f(pw_code(s2), code_c)

            m_init = in_progress_f(pw_code(comp), code_c)
            _, comp, _ = jax.lax.while_loop(
                cond, body, (jnp.int32(0), comp, m_init))

            for r in range(_R):
                cols = pl.ds(r * 128, 128)
                scat = jax.lax.dot_general(
                    comp, ohs[r], (((1,), (0,)), ((), ())),
                    preferred_element_type=f32)
                memb = jax.lax.dot_general(
                    ones1c, ohs[r], (((1,), (0,)), ((), ())),
                    preferred_element_type=f32)
                st_ref[:, cols] = jnp.where(memb > 0.0, scat,
                                            st_ref[:, cols])
            return start + _C

        jax.lax.while_loop(lambda s: s < n_act, chunk_body, jnp.int32(0))

    for r in range(_R):
        out_ref[pl.ds(r * 128, 128), :] = tr(st_ref[:, pl.ds(r * 128, 128)])


@jax.jit
def kernel(state, action, W_t, b_t, A_emb, W_p):
    act2 = action.reshape(1, _B)
    bt1 = b_t.reshape(_D, 1)
    return pl.pallas_call(
        _rollout_body,
        grid=(1,),
        in_specs=[
            pl.BlockSpec((_B, _D), lambda i: (0, 0)),
            pl.BlockSpec((1, _B), lambda i: (0, 0)),
            pl.BlockSpec((_D, _D), lambda i: (0, 0)),
            pl.BlockSpec((_D, 1), lambda i: (0, 0)),
            pl.BlockSpec((_A, _D), lambda i: (0, 0)),
            pl.BlockSpec((_D, _A), lambda i: (0, 0)),
        ],
        out_specs=pl.BlockSpec((_B, _D), lambda i: (0, 0)),
        out_shape=jax.ShapeDtypeStruct((_B, _D), jnp.float32),
        scratch_shapes=[
            pltpu.VMEM((_D, _B), jnp.float32),
            pltpu.VMEM((_D, _B), jnp.float32),
        ],
    )(state, act2, W_t, bt1, A_emb, W_p)


# speedup vs baseline: 8.5829x; 1.2986x over previous
import jax
import jax.numpy as jnp
from jax.experimental import pallas as pl
from jax.experimental.pallas import tpu as pltpu

_B, _D, _A = 2048, 128, 512
_MAX_T = 17
_HORIZON = 2
_CG = 256
_NCG = _B // _CG
_C = 256
_R = _B // 128


def _rollout_body(state_ref, act_ref, wt_ref, bt_ref, aemb_ref, wp_ref,
                  out_ref, st_ref, emb0_ref):
    f32 = jnp.float32
    i32 = jnp.int32
    W_t = wt_ref[...]
    b_t1 = bt_ref[...]
    A_emb = aemb_ref[...]
    W_p = wp_ref[...]

    eye128 = (jax.lax.broadcasted_iota(i32, (128, 128), 0) ==
              jax.lax.broadcasted_iota(i32, (128, 128), 1)).astype(f32)
    cum_u = (jax.lax.broadcasted_iota(i32, (128, 128), 0) <=
             jax.lax.broadcasted_iota(i32, (128, 128), 1)).astype(f32)
    excl_l = (jax.lax.broadcasted_iota(i32, (_R, _R), 1) <
              jax.lax.broadcasted_iota(i32, (_R, _R), 0)).astype(f32)
    ones128 = jnp.ones((128, 1), f32)
    ones1c = jnp.ones((1, _C), f32)
    k_iota = jax.lax.broadcasted_iota(i32, (_C, 128), 0).astype(f32)
    sub_ac = jax.lax.broadcasted_iota(i32, (_A, _C), 0)
    sub_ag = jax.lax.broadcasted_iota(i32, (_A, _CG), 0)

    def tr(x):
        return jax.lax.dot_general(x, eye128, (((0,), (0,)), ((), ())),
                                   preferred_element_type=f32)

    def argfirst0(x, iota, width):
        mx = jnp.max(x, axis=0, keepdims=True)
        return jnp.min(jnp.where(x == mx, iota, width), axis=0,
                       keepdims=True)

    def pw_code(sT):
        p = (sT[1:2, :] > sT[0:1, :]).astype(f32)
        a, b, c = sT[2:3, :], sT[3:4, :], sT[4:5, :]
        w1 = (b > a) & (b >= c)
        w2 = (c > a) & (c > b)
        w = jnp.where(w1, f32(1.0), f32(0.0)) + jnp.where(w2, f32(2.0),
                                                          f32(0.0))
        return 4.0 * p + w

    def in_progress_f(code_now, code_init):
        return (jnp.abs(code_now - code_init) == 4.0).astype(i32)

    def transition(sT, embT):
        return jax.nn.sigmoid(
            jax.lax.dot_general(W_t, sT, (((0,), (0,)), ((), ())),
                                preferred_element_type=f32) + embT + b_t1)

    for r in range(_R):
        st_ref[:, pl.ds(r * 128, 128)] = tr(state_ref[pl.ds(r * 128, 128), :])
    for g in range(_NCG):
        cols = pl.ds(g * _CG, _CG)
        oh0 = (sub_ag == act_ref[:, cols]).astype(f32)
        emb0_ref[:, cols] = jax.lax.dot_general(
            A_emb, oh0, (((0,), (0,)), ((), ())), preferred_element_type=f32)

    for _step in range(_HORIZON):
        code_gs, m_gs = [], []
        for g in range(_NCG):
            cols = pl.ds(g * _CG, _CG)
            sT = st_ref[:, cols]
            code_g = pw_code(sT)
            sT = transition(sT, emb0_ref[:, cols])
            m_gs.append(in_progress_f(pw_code(sT), code_g).astype(f32))
            st_ref[:, cols] = sT
            code_gs.append(code_g)
        code_full = jnp.concatenate(code_gs, axis=1)
        m_full = jnp.concatenate(m_gs, axis=1)

        def defl_cond(c):
            t0, m = c
            return (t0 < _MAX_T) & (jnp.sum(m) > f32(_C))

        def defl_body(c):
            t0, m = c
            new_ms = []
            for g in range(_NCG):
                cols = pl.ds(g * _CG, _CG)
                sT = st_ref[:, cols]
                p_logits = jax.lax.dot_general(
                    W_p, sT, (((0,), (0,)), ((), ())),
                    preferred_element_type=f32)
                aidx = argfirst0(p_logits, sub_ag, _A)
                oh = (sub_ag == aidx).astype(f32)
                emb = jax.lax.dot_general(
                    A_emb, oh, (((0,), (0,)), ((), ())),
                    preferred_element_type=f32)
                ns = transition(sT, emb)
                s2 = jnp.where(m[:, g * _CG:(g + 1) * _CG] > 0, ns, sT)
                st_ref[:, cols] = s2
                new_ms.append(in_progress_f(
                    pw_code(s2),
                    code_full[:, g * _CG:(g + 1) * _CG]).astype(f32))
            return t0 + 1, jnp.concatenate(new_ms, axis=1)

        t0, m_full = jax.lax.while_loop(defl_cond, defl_body,
                                        (jnp.int32(0), m_full))

        m_lane = jnp.concatenate(
            [m_full[:, r * 128:(r + 1) * 128] for r in range(_R)], axis=0)
        c_within = jnp.dot(m_lane, cum_u, preferred_element_type=f32)
        tot = jnp.dot(m_lane, ones128, preferred_element_type=f32)
        offs = jax.lax.dot_general(excl_l, tot, (((1,), (0,)), ((), ())),
                                   preferred_element_type=f32)
        pos_lane = c_within + offs - 1.0
        n_act = jnp.sum(tot).astype(i32)

        def make_onehots(start):
            s_f = start.astype(f32)
            ohs = []
            for r in range(_R):
                rel = pos_lane[r:r + 1, :] - s_f
                hit = (k_iota == rel) & (m_lane[r:r + 1, :] > 0.0)
                ohs.append(hit.astype(f32))
            return ohs

        def chunk_body(start):
            ohs = make_onehots(start)
            comp = None
            code_c = None
            for r in range(_R):
                cols = pl.ds(r * 128, 128)
                pc = jax.lax.dot_general(
                    st_ref[:, cols], ohs[r], (((1,), (1,)), ((), ())),
                    preferred_element_type=f32)
                cc = jax.lax.dot_general(
                    code_full[:, r * 128:(r + 1) * 128], ohs[r],
                    (((1,), (1,)), ((), ())), preferred_element_type=f32)
                comp = pc if comp is None else comp + pc
                code_c = cc if code_c is None else code_c + cc

            def cond(c):
                t, _s, m = c
                return (t < _MAX_T) & (jnp.max(m) > 0)

            def body(c):
                t, sT, m = c
                p_logits = jax.lax.dot_general(
                    W_p, sT, (((0,), (0,)), ((), ())),
                    preferred_element_type=f32)
                aidx = argfirst0(p_logits, sub_ac, _A)
                oh = (sub_ac == aidx).astype(f32)
                emb = jax.lax.dot_general(
                    A_emb, oh, (((0,), (0,)), ((), ())),
                    preferred_element_type=f32)
                ns = transition(sT, emb)
                s2 = jnp.where(m > 0, ns, sT)
                return t + 1, s2, in_progress_f(pw_code(s2), code_c)

            m_init = in_progress_f(pw_code(comp), code_c)
            _, comp, _ = jax.lax.while_loop(
                cond, body, (t0, comp, m_init))

            for r in range(_R):
                cols = pl.ds(r * 128, 128)
                scat = jax.lax.dot_general(
                    comp, ohs[r], (((1,), (0,)), ((), ())),
                    preferred_element_type=f32)
                memb = jax.lax.dot_general(
                    ones1c, ohs[r], (((1,), (0,)), ((), ())),
                    preferred_element_type=f32)
                st_ref[:, cols] = jnp.where(memb > 0.0, scat,
                                            st_ref[:, cols])
            return start + _C

        jax.lax.while_loop(lambda s: (s < n_act) & (t0 < _MAX_T),
                           chunk_body, jnp.int32(0))

    for r in range(_R):
        out_ref[pl.ds(r * 128, 128), :] = tr(st_ref[:, pl.ds(r * 128, 128)])


@jax.jit
def kernel(state, action, W_t, b_t, A_emb, W_p):
    act2 = action.reshape(1, _B)
    bt1 = b_t.reshape(_D, 1)
    return pl.pallas_call(
        _rollout_body,
        grid=(1,),
        in_specs=[
            pl.BlockSpec((_B, _D), lambda i: (0, 0)),
            pl.BlockSpec((1, _B), lambda i: (0, 0)),
            pl.BlockSpec((_D, _D), lambda i: (0, 0)),
            pl.BlockSpec((_D, 1), lambda i: (0, 0)),
            pl.BlockSpec((_A, _D), lambda i: (0, 0)),
            pl.BlockSpec((_D, _A), lambda i: (0, 0)),
        ],
        out_specs=pl.BlockSpec((_B, _D), lambda i: (0, 0)),
        out_shape=jax.ShapeDtypeStruct((_B, _D), jnp.float32),
        scratch_shapes=[
            pltpu.VMEM((_D, _B), jnp.float32),
            pltpu.VMEM((_D, _B), jnp.float32),
        ],
    )(state, act2, W_t, bt1, A_emb, W_p)
